# baseline (device time: 265975 ns/iter reference)
import jax
import jax.numpy as jnp
from jax import lax
from jax.experimental import pallas as pl
from jax.experimental.pallas import tpu as pltpu

N_DEV = 16
M = 4096
N = 2048
CH = M // N_DEV
QCH = M // (4 * N_DEV)
N_RINGS = 4
F8 = jnp.float8_e4m3fn


def kernel(x, w_mat):
    def body(x_ref, w_ref, out_ref, ag_buf,
             rcv0, rcv1, rcv2, rcv3, acc0, acc1, acc2, acc3,
             mx_send, mx_rcv,
             rs_send, rs_recv, ag_send_f, ag_recv_f, ag_send_b, ag_recv_b,
             mx_send_sems, mx_recv_sems, credit):
        my = lax.axis_index("i")
        left = lax.rem(my + N_DEV - 1, N_DEV)
        right = lax.rem(my + 1, N_DEV)

        rcvs = [rcv0, rcv1, rcv2, rcv3]
        accs = [acc0, acc1, acc2, acc3]
        rings = [
            (0 * M // 4, +1, right, left),
            (2 * M // 4, -1, left, right),
            (1 * M // 4, +1, right, left),
            (3 * M // 4, -1, left, right),
        ]

        def rows(r, k):
            return pl.ds(rings[r][0] + k * QCH, QCH)

        def chunk_idx(dir_, off, s):
            return lax.rem(my + 2 * N_DEV + dir_ * off - dir_ * s, N_DEV)

        barrier_sem = pltpu.get_barrier_semaphore()
        pl.semaphore_signal(barrier_sem, inc=1, device_id=(left,),
                            device_id_type=pl.DeviceIdType.MESH)
        pl.semaphore_signal(barrier_sem, inc=1, device_id=(right,),
                            device_id_type=pl.DeviceIdType.MESH)

        g_own = lax.div(my, 4)

        def gemm_group(r, g):
            sl = pl.ds(rings[r][0] + g * (4 * QCH), 4 * QCH)
            out_ref[sl, :] = jnp.dot(x_ref[sl, :], w_ref[...],
                                     preferred_element_type=jnp.float32)

        for r in range(N_RINGS):
            gemm_group(r, g_own)

        pl.semaphore_wait(barrier_sem, 2)

        def rs_rdma(r, s, src):
            _, _, to, _ = rings[r]
            return pltpu.make_async_remote_copy(
                src_ref=src, dst_ref=rcvs[r].at[s % 2],
                send_sem=rs_send.at[r], recv_sem=rs_recv.at[r, s % 2],
                device_id=(to,), device_id_type=pl.DeviceIdType.MESH)

        inflight = []
        for r in range(N_RINGS):
            rd = rs_rdma(r, 0, out_ref.at[rows(r, my), :])
            rd.start()
            inflight.append(rd)

        for t in range(1, 4):
            g = lax.rem(g_own + t, 4)
            for r in range(N_RINGS):
                gemm_group(r, g)

        for s in range(N_DEV - 1):
            for r in range(N_RINGS):
                _, dir_, _, frm = rings[r]
                inflight[r].wait()
                c = chunk_idx(dir_, 0, s + 1)
                accs[r][...] = rcvs[r][s % 2] + out_ref[rows(r, c), :]
                if s < N_DEV - 3:
                    pl.semaphore_signal(
                        credit.at[r], inc=1, device_id=(frm,),
                        device_id_type=pl.DeviceIdType.MESH)
                if s < N_DEV - 2:
                    if s >= 1:
                        pl.semaphore_wait(credit.at[r], 1)
                    rd = rs_rdma(r, s + 1, accs[r])
                    rd.start()
                    inflight[r] = rd

        my_max = jnp.float32(0.0)
        for r in range(N_RINGS):
            my_max = jnp.maximum(my_max, jnp.max(accs[r][...]))
        mx_send[...] = jnp.full((8, 128), my_max, jnp.float32)

        mx_sends = []
        for k in range(1, N_DEV):
            dst = lax.rem(my + k, N_DEV)
            rd = pltpu.make_async_remote_copy(
                src_ref=mx_send, dst_ref=mx_rcv.at[my],
                send_sem=mx_send_sems.at[k - 1], recv_sem=mx_recv_sems.at[my],
                device_id=(dst,), device_id_type=pl.DeviceIdType.MESH)
            rd.start()
            mx_sends.append(rd)
        amax = my_max
        for k in range(1, N_DEV):
            src = lax.rem(my + k, N_DEV)
            rd = pltpu.make_async_remote_copy(
                src_ref=mx_send, dst_ref=mx_rcv.at[src],
                send_sem=mx_send_sems.at[k - 1], recv_sem=mx_recv_sems.at[src],
                device_id=(left,), device_id_type=pl.DeviceIdType.MESH)
            rd.wait_recv()
            amax = jnp.maximum(amax, mx_rcv[src, 0, 0])
        for rd in mx_sends:
            rd.wait_send()

        scale = amax / 448.0

        for r in range(N_RINGS):
            _, dir_, _, _ = rings[r]
            own = chunk_idx(dir_, 1, 0)
            q = (jnp.maximum(accs[r][...], 0.0) / scale).astype(F8)
            ag_buf[rows(r, own), :] = q
            out_ref[rows(r, own), :] = q.astype(jnp.float32) * scale

        F_STEPS = N_DEV // 2
        B_STEPS = N_DEV // 2 - 1

        def ag_rdma(r, fwd, s):
            _, dir_, fwd_to, bwd_to = rings[r]
            if fwd:
                c = chunk_idx(dir_, 1, s)
                to, send, recv = fwd_to, ag_send_f.at[r], ag_recv_f.at[r, s]
            else:
                c = chunk_idx(-dir_, 0, s + 1)
                to, send, recv = bwd_to, ag_send_b.at[r], ag_recv_b.at[r, s]
            return pltpu.make_async_remote_copy(
                src_ref=ag_buf.at[rows(r, c), :],
                dst_ref=ag_buf.at[rows(r, c), :],
                send_sem=send, recv_sem=recv,
                device_id=(to,), device_id_type=pl.DeviceIdType.MESH)

        infF, infB = [], []
        for r in range(N_RINGS):
            rd = ag_rdma(r, True, 0)
            rd.start()
            infF.append(rd)
            rd = ag_rdma(r, False, 0)
            rd.start()
            infB.append(rd)

        for s in range(F_STEPS):
            for r in range(N_RINGS):
                _, dir_, _, _ = rings[r]
                infF[r].wait()
                if s < F_STEPS - 1:
                    rd = ag_rdma(r, True, s + 1)
                    rd.start()
                    infF[r] = rd
                c = chunk_idx(dir_, 0, s)
                out_ref[rows(r, c), :] = (
                    ag_buf[rows(r, c), :].astype(jnp.float32) * scale)
            if s < B_STEPS:
                for r in range(N_RINGS):
                    _, dir_, _, _ = rings[r]
                    infB[r].wait()
                    if s < B_STEPS - 1:
                        rd = ag_rdma(r, False, s + 1)
                        rd.start()
                        infB[r] = rd
                    c = chunk_idx(-dir_, 0, s + 2)
                    out_ref[rows(r, c), :] = (
                        ag_buf[rows(r, c), :].astype(jnp.float32) * scale)

    return pl.pallas_call(
        body,
        out_shape=jax.ShapeDtypeStruct((M, N), jnp.float32),
        in_specs=[
            pl.BlockSpec(memory_space=pltpu.VMEM),
            pl.BlockSpec(memory_space=pltpu.VMEM),
        ],
        out_specs=pl.BlockSpec(memory_space=pltpu.VMEM),
        scratch_shapes=(
            [pltpu.VMEM((M, N), F8)] +
            [pltpu.VMEM((2, QCH, N), jnp.float32)] * 4 +
            [pltpu.VMEM((QCH, N), jnp.float32)] * 4 +
            [
                pltpu.VMEM((8, 128), jnp.float32),
                pltpu.VMEM((N_DEV, 8, 128), jnp.float32),
                pltpu.SemaphoreType.DMA((N_RINGS,)),
                pltpu.SemaphoreType.DMA((N_RINGS, 2)),
                pltpu.SemaphoreType.DMA((N_RINGS,)),
                pltpu.SemaphoreType.DMA((N_RINGS, N_DEV // 2)),
                pltpu.SemaphoreType.DMA((N_RINGS,)),
                pltpu.SemaphoreType.DMA((N_RINGS, N_DEV // 2 - 1)),
                pltpu.SemaphoreType.DMA((N_DEV - 1,)),
                pltpu.SemaphoreType.DMA((N_DEV,)),
                pltpu.SemaphoreType.REGULAR((N_RINGS,)),
            ]
        ),
        compiler_params=pltpu.CompilerParams(
            collective_id=0, vmem_limit_bytes=64 * 1024 * 1024),
    )(x, w_mat)
